# Initial kernel scaffold; baseline (speedup 1.0000x reference)
import jax
import jax.numpy as jnp
from jax.experimental import pallas as pl


def kernel(inputs):
    raise NotImplementedError("write your pallas kernel here")


# speedup vs baseline: 1.6820x; 1.6820x over previous
import functools

import jax
import jax.numpy as jnp
import numpy as np
from jax import lax
from jax.experimental import pallas as pl
from jax.experimental.pallas import tpu as pltpu
from jax.experimental.pallas import tpu_sc as plsc

_ROWS = 64
_COLS = 8192
_KEEP = 256
_LANES = 16
_VECS = _COLS // _LANES
_NC = 2
_NS = 16
_ROWS_PER_W = _ROWS // (_NC * _NS)

_INT_MIN = np.int32(-2147483648)


def _order_key(b):
    return jnp.where(b >= 0, b, jnp.bitwise_xor(jnp.bitwise_not(b), _INT_MIN))


def _kwta_body(in_hbm, out_hbm, row_v, key_v):
    wid = lax.axis_index("s") * _NC + lax.axis_index("c")

    for r in range(_ROWS_PER_W):
        row = wid * _ROWS_PER_W + r
        base = row * _COLS
        pltpu.sync_copy(in_hbm.at[pl.ds(base, _COLS)], row_v)

        def map_body(j, carry):
            x16 = row_v[pl.ds(j * _LANES, _LANES)]
            b16 = lax.bitcast_convert_type(x16, jnp.int32)
            key_v[pl.ds(j * _LANES, _LANES)] = _order_key(b16)
            return carry

        lax.fori_loop(0, _VECS, map_body, np.int32(0))

        def bit_body(i, acc):
            trial = acc + (np.int32(1) << (np.int32(31) - i))

            def cnt_body(j, c16):
                k16 = key_v[pl.ds(j * _LANES, _LANES)]
                return c16 + jnp.where(k16 >= trial, np.int32(1),
                                       np.int32(0))

            c16 = lax.fori_loop(0, _VECS, cnt_body,
                                jnp.zeros((_LANES,), jnp.int32))
            cnt = c16[0]
            for lane in range(1, _LANES):
                cnt = cnt + c16[lane]
            return jnp.where(cnt >= _KEEP, trial, acc)

        thr = lax.fori_loop(0, 32, bit_body, _INT_MIN)

        def mask_body(j, carry):
            x16 = row_v[pl.ds(j * _LANES, _LANES)]
            k16 = key_v[pl.ds(j * _LANES, _LANES)]
            row_v[pl.ds(j * _LANES, _LANES)] = jnp.where(
                k16 >= thr, x16, np.float32(0.0))
            return carry

        lax.fori_loop(0, _VECS, mask_body, np.int32(0))

        pltpu.sync_copy(row_v, out_hbm.at[pl.ds(base, _COLS)])


@functools.partial(jax.jit, static_argnums=())
def _kwta(flat):
    mesh = plsc.VectorSubcoreMesh(core_axis_name="c", subcore_axis_name="s")
    fn = functools.partial(
        pl.kernel,
        mesh=mesh,
        out_type=jax.ShapeDtypeStruct((_ROWS * _COLS,), jnp.float32),
        scratch_types=[
            pltpu.VMEM((_COLS,), jnp.float32),
            pltpu.VMEM((_COLS,), jnp.int32),
        ],
    )(_kwta_body)
    return fn(flat)


def kernel(inputs):
    out_flat = _kwta(inputs.reshape(-1))
    return out_flat.reshape(inputs.shape)


# speedup vs baseline: 5.1380x; 3.0547x over previous
import functools

import jax
import jax.numpy as jnp
import numpy as np
from jax import lax
from jax.experimental import pallas as pl
from jax.experimental.pallas import tpu as pltpu
from jax.experimental.pallas import tpu_sc as plsc

_ROWS = 64
_COLS = 8192
_KEEP = 256
_LANES = 16
_VECS = _COLS // _LANES
_NC = 2
_NS = 16
_ROWS_PER_W = _ROWS // (_NC * _NS)
_UNROLL = 8

_INT_MIN = np.int32(-2147483648)


def _order_key(b):
    return jnp.where(b >= 0, b, jnp.bitwise_xor(jnp.bitwise_not(b), _INT_MIN))


def _kwta_body(in_hbm, out_hbm, row_v, key_v):
    wid = lax.axis_index("s") * _NC + lax.axis_index("c")

    for r in range(_ROWS_PER_W):
        row = wid * _ROWS_PER_W + r
        base = row * _COLS
        pltpu.sync_copy(in_hbm.at[pl.ds(base, _COLS)], row_v)

        def map_body(j, carry):
            base16 = j * (_LANES * _UNROLL)
            for u in range(_UNROLL):
                x16 = row_v[pl.ds(base16 + u * _LANES, _LANES)]
                b16 = lax.bitcast_convert_type(x16, jnp.int32)
                key_v[pl.ds(base16 + u * _LANES, _LANES)] = _order_key(b16)
            return carry

        lax.fori_loop(0, _VECS // _UNROLL, map_body, np.int32(0))

        def bit_body(i, acc):
            trial = acc + (np.int32(1) << (np.int32(31) - i))

            def cnt_body(j, cs):
                base16 = j * (_LANES * _UNROLL)
                return tuple(
                    cs[u] + jnp.where(
                        key_v[pl.ds(base16 + u * _LANES, _LANES)] >= trial,
                        np.int32(1), np.int32(0))
                    for u in range(_UNROLL))

            cs = lax.fori_loop(
                0, _VECS // _UNROLL, cnt_body,
                tuple(jnp.zeros((_LANES,), jnp.int32)
                      for _ in range(_UNROLL)))
            c16 = cs[0]
            for u in range(1, _UNROLL):
                c16 = c16 + cs[u]
            cnt = c16[0]
            for lane in range(1, _LANES):
                cnt = cnt + c16[lane]
            return jnp.where(cnt >= _KEEP, trial, acc)

        thr = lax.fori_loop(0, 32, bit_body, _INT_MIN)

        def mask_body(j, carry):
            base16 = j * (_LANES * _UNROLL)
            for u in range(_UNROLL):
                x16 = row_v[pl.ds(base16 + u * _LANES, _LANES)]
                k16 = key_v[pl.ds(base16 + u * _LANES, _LANES)]
                row_v[pl.ds(base16 + u * _LANES, _LANES)] = jnp.where(
                    k16 >= thr, x16, np.float32(0.0))
            return carry

        lax.fori_loop(0, _VECS // _UNROLL, mask_body, np.int32(0))

        pltpu.sync_copy(row_v, out_hbm.at[pl.ds(base, _COLS)])


@functools.partial(jax.jit, static_argnums=())
def _kwta(flat):
    mesh = plsc.VectorSubcoreMesh(core_axis_name="c", subcore_axis_name="s")
    fn = functools.partial(
        pl.kernel,
        mesh=mesh,
        out_type=jax.ShapeDtypeStruct((_ROWS * _COLS,), jnp.float32),
        scratch_types=[
            pltpu.VMEM((_COLS,), jnp.float32),
            pltpu.VMEM((_COLS,), jnp.int32),
        ],
    )(_kwta_body)
    return fn(flat)


def kernel(inputs):
    out_flat = _kwta(inputs.reshape(-1))
    return out_flat.reshape(inputs.shape)
